# baseline (speedup 1.0000x reference)
import math

import jax
import jax.numpy as jnp
from jax.experimental import pallas as pl
from jax.experimental.pallas import tpu as pltpu

_TAU = 0.1
_EPS = 1e-10
_N = 4096
_BR = 256
_NB = _N // _BR
_K2 = math.log2(math.e) / _TAU


def _rowsum_mxu(m):
    ones = jnp.ones((m.shape[1], 1), dtype=m.dtype)
    return jax.lax.dot_general(m, ones, (((1,), (0,)), ((), ())),
                               preferred_element_type=jnp.float32)


def _colsum_mxu(m):
    ones = jnp.ones((1, m.shape[0]), dtype=m.dtype)
    return jax.lax.dot_general(ones, m, (((1,), (0,)), ((), ())),
                               preferred_element_type=jnp.float32)


def _diag_sum(t_d):
    rows = jax.lax.broadcasted_iota(jnp.int32, (_BR, _BR), 0)
    cols = jax.lax.broadcasted_iota(jnp.int32, (_BR, _BR), 1)
    return jnp.sum(jnp.where(rows == cols, t_d, 0.0))


def _ccl_kernel(x_ref, out_ref, ebcache_ref, colsum_ref):
    p = pl.program_id(0)
    i = pl.program_id(1)

    @pl.when(p == 0)
    def _phase0():
        e = jnp.exp2(x_ref[...] * _K2)
        eb = e.astype(jnp.bfloat16)
        ebcache_ref[pl.ds(i * _BR, _BR), :] = eb

        r = _rowsum_mxu(e) + _EPS
        part = _colsum_mxu(eb)

        @pl.when(i == 0)
        def _():
            colsum_ref[...] = part

        @pl.when(i > 0)
        def _():
            colsum_ref[...] = colsum_ref[...] + part

        a = jnp.maximum(1.0 - e / r, 0.0) + _EPS
        t = jnp.log(a)
        s_full = jnp.sum(_colsum_mxu(t))

        e_d = jnp.exp2(x_ref[:, pl.ds(i * _BR, _BR)] * _K2)
        a_d = jnp.maximum(1.0 - e_d / r, 0.0) + _EPS
        s = s_full - _diag_sum(jnp.log(a_d))

        @pl.when(i == 0)
        def _():
            out_ref[0, 0] = s

        @pl.when(i > 0)
        def _():
            out_ref[0, 0] = out_ref[0, 0] + s

    @pl.when(p == 1)
    def _phase1():
        e = ebcache_ref[pl.ds(i * _BR, _BR), :].astype(jnp.float32)
        c = colsum_ref[...] + _EPS
        b = jnp.maximum(1.0 - e / c, 0.0) + _EPS
        t = jnp.log(b)
        s_full = jnp.sum(_colsum_mxu(t))

        e_d = ebcache_ref[pl.ds(i * _BR, _BR),
                          pl.ds(i * _BR, _BR)].astype(jnp.float32)
        c_d = colsum_ref[:, pl.ds(i * _BR, _BR)] + _EPS
        b_d = jnp.maximum(1.0 - e_d / c_d, 0.0) + _EPS
        s = s_full - _diag_sum(jnp.log(b_d))

        out_ref[0, 0] = out_ref[0, 0] + s

        @pl.when(i == _NB - 1)
        def _():
            out_ref[0, 0] = out_ref[0, 0] * (-1.0 / _N)


def kernel(scores):
    out = pl.pallas_call(
        _ccl_kernel,
        grid=(2, _NB),
        in_specs=[pl.BlockSpec((_BR, _N),
                               lambda p, i: (jnp.where(p == 0, i, _NB - 1), 0))],
        out_specs=pl.BlockSpec((1, 1), lambda p, i: (0, 0), memory_space=pltpu.SMEM),
        out_shape=jax.ShapeDtypeStruct((1, 1), jnp.float32),
        scratch_shapes=[
            pltpu.VMEM((_N, _N), jnp.bfloat16),
            pltpu.VMEM((1, _N), jnp.float32),
        ],
        compiler_params=pltpu.CompilerParams(
            vmem_limit_bytes=62 * 1024 * 1024,
        ),
    )(scores)
    return out[0, 0]


# speedup vs baseline: 1.0820x; 1.0820x over previous
import math

import jax
import jax.numpy as jnp
from jax.experimental import pallas as pl
from jax.experimental.pallas import tpu as pltpu

_TAU = 0.1
_EPS = 1e-10
_N = 4096
_BR = 512
_NB = _N // _BR
_K2 = math.log2(math.e) / _TAU


def _rowsum_mxu(m):
    ones = jnp.ones((m.shape[1], 1), dtype=m.dtype)
    return jax.lax.dot_general(m, ones, (((1,), (0,)), ((), ())),
                               preferred_element_type=jnp.float32)


def _colsum_mxu(m):
    ones = jnp.ones((1, m.shape[0]), dtype=m.dtype)
    return jax.lax.dot_general(ones, m, (((1,), (0,)), ((), ())),
                               preferred_element_type=jnp.float32)


def _diag_sum(t_d):
    rows = jax.lax.broadcasted_iota(jnp.int32, (_BR, _BR), 0)
    cols = jax.lax.broadcasted_iota(jnp.int32, (_BR, _BR), 1)
    return jnp.sum(jnp.where(rows == cols, t_d, 0.0))


def _ccl_kernel(x_ref, out_ref, ebcache_ref, colsum_ref):
    p = pl.program_id(0)
    i = pl.program_id(1)

    @pl.when(p == 0)
    def _phase0():
        e = jnp.exp2(x_ref[...] * _K2)
        eb = e.astype(jnp.bfloat16)
        ebcache_ref[pl.ds(i * _BR, _BR), :] = eb

        r = _rowsum_mxu(e) + _EPS
        part = _colsum_mxu(eb)

        @pl.when(i == 0)
        def _():
            colsum_ref[...] = part

        @pl.when(i > 0)
        def _():
            colsum_ref[...] = colsum_ref[...] + part

        a = jnp.maximum(1.0 - e / r, 0.0) + _EPS
        t = jnp.log(a)
        s_full = jnp.sum(_colsum_mxu(t))

        e_d = jnp.exp2(x_ref[:, pl.ds(i * _BR, _BR)] * _K2)
        a_d = jnp.maximum(1.0 - e_d / r, 0.0) + _EPS
        s = s_full - _diag_sum(jnp.log(a_d))

        @pl.when(i == 0)
        def _():
            out_ref[0, 0] = s

        @pl.when(i > 0)
        def _():
            out_ref[0, 0] = out_ref[0, 0] + s

    @pl.when(p == 1)
    def _phase1():
        e = ebcache_ref[pl.ds(i * _BR, _BR), :].astype(jnp.float32)
        c = colsum_ref[...] + _EPS
        b = jnp.maximum(1.0 - e / c, 0.0) + _EPS
        t = jnp.log(b)
        s_full = jnp.sum(_colsum_mxu(t))

        e_d = ebcache_ref[pl.ds(i * _BR, _BR),
                          pl.ds(i * _BR, _BR)].astype(jnp.float32)
        c_d = colsum_ref[:, pl.ds(i * _BR, _BR)] + _EPS
        b_d = jnp.maximum(1.0 - e_d / c_d, 0.0) + _EPS
        s = s_full - _diag_sum(jnp.log(b_d))

        out_ref[0, 0] = out_ref[0, 0] + s

        @pl.when(i == _NB - 1)
        def _():
            out_ref[0, 0] = out_ref[0, 0] * (-1.0 / _N)


def kernel(scores):
    out = pl.pallas_call(
        _ccl_kernel,
        grid=(2, _NB),
        in_specs=[pl.BlockSpec((_BR, _N),
                               lambda p, i: (jnp.where(p == 0, i, _NB - 1), 0))],
        out_specs=pl.BlockSpec((1, 1), lambda p, i: (0, 0), memory_space=pltpu.SMEM),
        out_shape=jax.ShapeDtypeStruct((1, 1), jnp.float32),
        scratch_shapes=[
            pltpu.VMEM((_N, _N), jnp.bfloat16),
            pltpu.VMEM((1, _N), jnp.float32),
        ],
        compiler_params=pltpu.CompilerParams(
            vmem_limit_bytes=62 * 1024 * 1024,
        ),
    )(scores)
    return out[0, 0]


# speedup vs baseline: 1.2469x; 1.1523x over previous
import math

import jax
import jax.numpy as jnp
from jax.experimental import pallas as pl
from jax.experimental.pallas import tpu as pltpu

_TAU = 0.1
_EPS = 1e-10
_N = 4096
_BR = 512
_NB = _N // _BR
_K2 = math.log2(math.e) / _TAU


def _rowsum_mxu(m):
    ones = jnp.ones((m.shape[1], 1), dtype=m.dtype)
    return jax.lax.dot_general(m, ones, (((1,), (0,)), ((), ())),
                               preferred_element_type=jnp.float32)


def _colsum_mxu(m):
    ones = jnp.ones((1, m.shape[0]), dtype=m.dtype)
    return jax.lax.dot_general(ones, m, (((1,), (0,)), ((), ())),
                               preferred_element_type=jnp.float32)


def _diag_sum(t_d):
    rows = jax.lax.broadcasted_iota(jnp.int32, (_BR, _BR), 0)
    cols = jax.lax.broadcasted_iota(jnp.int32, (_BR, _BR), 1)
    return jnp.sum(jnp.where(rows == cols, t_d, 0.0))


def _ccl_kernel(x_ref, out_ref, ebcache_ref, colsum_ref):
    p = pl.program_id(0)
    i = pl.program_id(1)

    @pl.when(p == 0)
    def _phase0():
        e = jnp.exp2(x_ref[...] * _K2)
        eb = e.astype(jnp.bfloat16)
        ebcache_ref[pl.ds(i * _BR, _BR), :] = eb

        r = jnp.sum(e, axis=1, keepdims=True) + _EPS
        part = _colsum_mxu(eb)

        @pl.when(i == 0)
        def _():
            colsum_ref[...] = part

        @pl.when(i > 0)
        def _():
            colsum_ref[...] = colsum_ref[...] + part

        rinv = 1.0 / r
        a = jnp.maximum(r - e, 0.0) * rinv + _EPS
        t = jnp.log(a)
        s_full = jnp.sum(_colsum_mxu(t))

        e_d = jnp.exp2(x_ref[:, pl.ds(i * _BR, _BR)] * _K2)
        a_d = jnp.maximum(r - e_d, 0.0) * rinv + _EPS
        s = s_full - _diag_sum(jnp.log(a_d))

        @pl.when(i == 0)
        def _():
            out_ref[0, 0] = s

        @pl.when(i > 0)
        def _():
            out_ref[0, 0] = out_ref[0, 0] + s

    @pl.when(p == 1)
    def _phase1():
        e = ebcache_ref[pl.ds(i * _BR, _BR), :].astype(jnp.float32)
        c = colsum_ref[...] + _EPS
        cinv = 1.0 / c
        b = jnp.maximum(c - e, 0.0) * cinv + _EPS
        t = jnp.log(b)
        s_full = jnp.sum(_colsum_mxu(t))

        e_d = ebcache_ref[pl.ds(i * _BR, _BR),
                          pl.ds(i * _BR, _BR)].astype(jnp.float32)
        c_d = colsum_ref[:, pl.ds(i * _BR, _BR)] + _EPS
        b_d = jnp.maximum(c_d - e_d, 0.0) * (1.0 / c_d) + _EPS
        s = s_full - _diag_sum(jnp.log(b_d))

        out_ref[0, 0] = out_ref[0, 0] + s

        @pl.when(i == _NB - 1)
        def _():
            out_ref[0, 0] = out_ref[0, 0] * (-1.0 / _N)


def kernel(scores):
    out = pl.pallas_call(
        _ccl_kernel,
        grid=(2, _NB),
        in_specs=[pl.BlockSpec((_BR, _N),
                               lambda p, i: (jnp.where(p == 0, i, _NB - 1), 0))],
        out_specs=pl.BlockSpec((1, 1), lambda p, i: (0, 0), memory_space=pltpu.SMEM),
        out_shape=jax.ShapeDtypeStruct((1, 1), jnp.float32),
        scratch_shapes=[
            pltpu.VMEM((_N, _N), jnp.bfloat16),
            pltpu.VMEM((1, _N), jnp.float32),
        ],
        compiler_params=pltpu.CompilerParams(
            vmem_limit_bytes=62 * 1024 * 1024,
        ),
    )(scores)
    return out[0, 0]
